# baseline (speedup 1.0000x reference)
import jax
import jax.numpy as jnp
from jax import lax
from jax.experimental import pallas as pl
from jax.experimental.pallas import tpu as pltpu
from jax.experimental.pallas import tpu_sc as plsc

_WIDTH = 2048
_HEIGHT = 2048
_N_UVS = 1048576

_NC = 2
_NS = 16
_NW = _NC * _NS
_CHUNK = 128
_PER_W = _N_UVS // _NW
_ITERS = _PER_W // _CHUNK
_L = 16


def _sc_body(uvs_hbm, tex_hbm, out_hbm,
             uv_v, idx00_v, idx10_v, idx01_v, idx11_v,
             a_v, b_v, cb0_v, cb1_v,
             c00_v, c10_v, c01_v, c11_v, out_v, sem):
    cid = lax.axis_index("c")
    sid = lax.axis_index("s")
    wid = sid * _NC + cid
    base = wid * _PER_W

    lane = lax.iota(jnp.int32, _L)
    lane2 = lane * 2
    lane3 = lane * 3

    def chunk_body(g, carry):
        off = pl.multiple_of(base + g * _CHUNK, _CHUNK)
        pltpu.sync_copy(uvs_hbm.at[pl.ds(off * 2, _CHUNK * 2)], uv_v)

        for i in range(_CHUNK // _L):
            us = plsc.load_gather(uv_v, [lane2 + (2 * _L * i)])
            vs = plsc.load_gather(uv_v, [lane2 + (2 * _L * i + 1)])
            u = ((us + 1.0) * 0.5) * (_WIDTH - 1)
            v = ((vs + 1.0) * 0.5) * (_HEIGHT - 1)
            u0 = u.astype(jnp.int32)
            v0 = v.astype(jnp.int32)
            af = u - u0.astype(jnp.float32)
            bf = v - v0.astype(jnp.float32)
            u1 = u0 + jnp.where(af > 0.0, 1, 0)
            v1 = v0 + jnp.where(bf > 0.0, 1, 0)
            r0 = u0 * 512
            r1 = u1 * 512
            vq0 = v0 >> 2
            vq1 = v1 >> 2
            sl = pl.ds(_L * i, _L)
            idx00_v[sl] = r0 + vq0
            idx10_v[sl] = r1 + vq0
            idx01_v[sl] = r0 + vq1
            idx11_v[sl] = r1 + vq1
            cb0_v[sl] = (v0 & 3) * 4
            cb1_v[sl] = (v1 & 3) * 4
            a_v[sl] = af
            b_v[sl] = bf

        cp0 = pltpu.async_copy(tex_hbm.at[idx00_v], c00_v, sem)
        cp1 = pltpu.async_copy(tex_hbm.at[idx10_v], c10_v, sem)
        cp2 = pltpu.async_copy(tex_hbm.at[idx01_v], c01_v, sem)
        cp3 = pltpu.async_copy(tex_hbm.at[idx11_v], c11_v, sem)
        cp0.wait()
        cp1.wait()
        cp2.wait()
        cp3.wait()

        for i in range(_CHUNK // _L):
            sl = pl.ds(_L * i, _L)
            af = a_v[sl]
            bf = b_v[sl]
            cb0 = cb0_v[sl]
            cb1 = cb1_v[sl]
            naf = 1.0 - af
            nbf = 1.0 - bf
            row16 = lane + (_L * i)
            obase = lane3 + (3 * _L * i)
            for ch in range(3):
                c00 = plsc.load_gather(c00_v, [row16, cb0 + ch])
                c10 = plsc.load_gather(c10_v, [row16, cb0 + ch])
                c01 = plsc.load_gather(c01_v, [row16, cb1 + ch])
                c11 = plsc.load_gather(c11_v, [row16, cb1 + ch])
                x = (c00 * af + c10 * naf) * bf + (c01 * af + c11 * naf) * nbf
                xc = jnp.minimum(jnp.maximum(x, -9.0), 9.0)
                e = jnp.exp(xc + xc)
                t = (e - 1.0) / (e + 1.0)
                plsc.store_scatter(out_v, [obase + ch], t)

        pltpu.sync_copy(out_v, out_hbm.at[pl.ds(off * 3, _CHUNK * 3)])
        return carry

    lax.fori_loop(0, _ITERS, chunk_body, 0)


def kernel(uvs, texture):
    texp = jnp.pad(texture.reshape(_HEIGHT * _WIDTH, 3), ((0, 0), (0, 1)))
    tex16 = texp.reshape(_HEIGHT * _WIDTH // 4, 16)
    kfn = pl.kernel(
        _sc_body,
        out_type=jax.ShapeDtypeStruct((_N_UVS * 3,), jnp.float32),
        mesh=plsc.VectorSubcoreMesh(
            core_axis_name="c", subcore_axis_name="s",
            num_cores=_NC, num_subcores=_NS),
        scratch_types=[
            pltpu.VMEM((_CHUNK * 2,), jnp.float32),
            pltpu.VMEM((_CHUNK,), jnp.int32),
            pltpu.VMEM((_CHUNK,), jnp.int32),
            pltpu.VMEM((_CHUNK,), jnp.int32),
            pltpu.VMEM((_CHUNK,), jnp.int32),
            pltpu.VMEM((_CHUNK,), jnp.float32),
            pltpu.VMEM((_CHUNK,), jnp.float32),
            pltpu.VMEM((_CHUNK,), jnp.int32),
            pltpu.VMEM((_CHUNK,), jnp.int32),
            pltpu.VMEM((_CHUNK, 16), jnp.float32),
            pltpu.VMEM((_CHUNK, 16), jnp.float32),
            pltpu.VMEM((_CHUNK, 16), jnp.float32),
            pltpu.VMEM((_CHUNK, 16), jnp.float32),
            pltpu.VMEM((_CHUNK * 3,), jnp.float32),
            pltpu.SemaphoreType.DMA,
        ],
        compiler_params=pltpu.CompilerParams(
            needs_layout_passes=False, use_tc_tiling_on_sc=False),
    )
    out = kfn(uvs.reshape(-1), tex16)
    return out.reshape(_N_UVS, 3)


# speedup vs baseline: 17.7732x; 17.7732x over previous
import jax
import jax.numpy as jnp
from jax import lax
from jax.experimental import pallas as pl
from jax.experimental.pallas import tpu as pltpu
from jax.experimental.pallas import tpu_sc as plsc

_WIDTH = 2048
_HEIGHT = 2048
_N_UVS = 1048576
_NTEX = _HEIGHT * _WIDTH
_NELEM = _NTEX * 4
_HALF = _NELEM
_NWIN = _NELEM // 16

_NC = 2
_NS = 16
_NW = _NC * _NS
_L = 16

_SZE = _NELEM // _NW
_BB = 16384
_TB = _BB // 4
_TSTAGE = _TB + 16
_BSTEPS = _SZE // _BB
_IVLEN = _BB + 64

_CHUNK = 128
_PER_W = _N_UVS // _NW
_ITERS = _PER_W // _CHUNK


def _build_body(p0_hbm, p1_hbm, p2_hbm, tbl_hbm,
                st0a, st1a, st2a, st0b, st1b, st2b, iva, ivb,
                in_a, in_b, out_a, out_b):
    wid = lax.axis_index("s") * _NC + lax.axis_index("c")
    ebase = wid * _SZE
    lane = lax.iota(jnp.int32, _L)
    lane4 = lane * 4
    planes = (p0_hbm, p1_hbm, p2_hbm)
    stages = ((st0a, st1a, st2a), (st0b, st1b, st2b))
    ivs = (iva, ivb)
    in_sems = (in_a, in_b)
    out_sems = (out_a, out_b)

    def fire_stage(j, k):
        t0 = pl.multiple_of((ebase + j * _BB) >> 2, 8)
        for c in range(3):
            pltpu.async_copy(planes[c].at[pl.ds(t0, _TSTAGE)],
                             stages[k][c], in_sems[k])

    def wait_stage(k):
        for c in range(3):
            pltpu.make_async_copy(planes[c].at[pl.ds(0, _TSTAGE)],
                                  stages[k][c], in_sems[k]).wait()

    def fire_out(j, k):
        eb = pl.multiple_of(ebase + j * _BB, 8)
        pltpu.async_copy(ivs[k].at[pl.ds(0, _BB)],
                         tbl_hbm.at[pl.ds(eb, _BB)], out_sems[k])
        pltpu.async_copy(ivs[k].at[pl.ds(8, _BB)],
                         tbl_hbm.at[pl.ds(_HALF + eb, _BB)], out_sems[k])

    def wait_out(k):
        for _ in range(2):
            pltpu.make_async_copy(ivs[k].at[pl.ds(0, _BB)],
                                  tbl_hbm.at[pl.ds(0, _BB)],
                                  out_sems[k]).wait()

    def interleave(k):
        for g in range(_TSTAGE // _L):
            gb = 64 * g
            for c in range(3):
                val = stages[k][c][pl.ds(_L * g, _L)]
                plsc.store_scatter(ivs[k], [lane4 + (gb + c)], val)

    fire_stage(0, 0)

    def superstep(ss, carry):
        for k in range(2):
            j = ss * 2 + k
            if k == 0:
                fire_stage(j + 1, 1)
            else:
                @pl.when(ss < _BSTEPS // 2 - 1)
                def _():
                    fire_stage(j + 1, 0)
            wait_stage(k)

            @pl.when(ss >= 1)
            def _():
                wait_out(k)

            interleave(k)
            fire_out(j, k)
        return carry

    lax.fori_loop(0, _BSTEPS // 2, superstep, 0)
    wait_out(0)
    wait_out(1)


def _gather_body(u_hbm, v_hbm, tbl_hbm, r_hbm, g_hbm, b_hbm,
                 u_v, v_v, rA_v, rB_v, cb_v, cbd_v, a_v, b_v,
                 cA_v, cB_v, or_v, og_v, ob_v, sem):
    wid = lax.axis_index("s") * _NC + lax.axis_index("c")
    base = wid * _PER_W
    lane = lax.iota(jnp.int32, _L)
    out_bufs = (or_v, og_v, ob_v)
    out_hbms = (r_hbm, g_hbm, b_hbm)

    def chunk_body(g, carry):
        off = pl.multiple_of(base + g * _CHUNK, _CHUNK)
        pltpu.sync_copy(u_hbm.at[pl.ds(off, _CHUNK)], u_v)
        pltpu.sync_copy(v_hbm.at[pl.ds(off, _CHUNK)], v_v)

        for i in range(_CHUNK // _L):
            sl = pl.ds(_L * i, _L)
            us = u_v[sl]
            vs = v_v[sl]
            u = ((us + 1.0) * 0.5) * (_WIDTH - 1)
            v = ((vs + 1.0) * 0.5) * (_HEIGHT - 1)
            u0 = u.astype(jnp.int32)
            v0 = v.astype(jnp.int32)
            af = u - u0.astype(jnp.float32)
            bf = v - v0.astype(jnp.float32)
            u1 = u0 + jnp.where(af > 0.0, 1, 0)
            dv = jnp.where(bf > 0.0, 1, 0)
            s00 = u0 * _WIDTH + v0
            s10 = u1 * _WIDTH + v0
            w0 = s00 >> 1
            w1 = s10 >> 1
            rA = (w0 >> 1) + ((w0 & 1) << 20)
            rB = (w1 >> 1) + ((w1 & 1) << 20)
            cb = (s00 & 1) * 4
            rA_v[sl] = rA
            rB_v[sl] = rB
            cb_v[sl] = cb
            cbd_v[sl] = cb + dv * 4
            a_v[sl] = af
            b_v[sl] = bf

        cp0 = pltpu.async_copy(tbl_hbm.at[rA_v], cA_v, sem)
        cp1 = pltpu.async_copy(tbl_hbm.at[rB_v], cB_v, sem)
        cp0.wait()
        cp1.wait()

        for i in range(_CHUNK // _L):
            sl = pl.ds(_L * i, _L)
            af = a_v[sl]
            bf = b_v[sl]
            cb = cb_v[sl]
            cbd = cbd_v[sl]
            naf = 1.0 - af
            nbf = 1.0 - bf
            row16 = lane + (_L * i)
            for ch in range(3):
                c00 = plsc.load_gather(cA_v, [row16, cb + ch])
                c01 = plsc.load_gather(cA_v, [row16, cbd + ch])
                c10 = plsc.load_gather(cB_v, [row16, cb + ch])
                c11 = plsc.load_gather(cB_v, [row16, cbd + ch])
                x = (c00 * af + c10 * naf) * bf + (c01 * af + c11 * naf) * nbf
                xc = jnp.minimum(jnp.maximum(x, -9.0), 9.0)
                e = jnp.exp(xc + xc)
                t = (e - 1.0) / (e + 1.0)
                out_bufs[ch][sl] = t

        for ch in range(3):
            pltpu.sync_copy(out_bufs[ch], out_hbms[ch].at[pl.ds(off, _CHUNK)])
        return carry

    lax.fori_loop(0, _ITERS, chunk_body, 0)


_MESH = dict(core_axis_name="c", subcore_axis_name="s",
             num_cores=_NC, num_subcores=_NS)
_CPARAMS = pltpu.CompilerParams(
    needs_layout_passes=False, use_tc_tiling_on_sc=False)


def kernel(uvs, texture):
    u = uvs[:, 0]
    v = uvs[:, 1]
    planes = [jnp.pad(texture[:, :, c].reshape(-1), (0, 16))
              for c in range(3)]

    build = pl.kernel(
        _build_body,
        out_type=jax.ShapeDtypeStruct((2 * _HALF,), jnp.float32),
        mesh=plsc.VectorSubcoreMesh(**_MESH),
        scratch_types=(
            [pltpu.VMEM((_TSTAGE,), jnp.float32)] * 6
            + [pltpu.VMEM((_IVLEN,), jnp.float32)] * 2
            + [pltpu.SemaphoreType.DMA] * 4),
        compiler_params=_CPARAMS,
    )
    tbl = build(*planes).reshape(2 * _NWIN, 16)

    gather = pl.kernel(
        _gather_body,
        out_type=(jax.ShapeDtypeStruct((_N_UVS,), jnp.float32),) * 3,
        mesh=plsc.VectorSubcoreMesh(**_MESH),
        scratch_types=(
            [pltpu.VMEM((_CHUNK,), jnp.float32)] * 2
            + [pltpu.VMEM((_CHUNK,), jnp.int32)] * 4
            + [pltpu.VMEM((_CHUNK,), jnp.float32)] * 2
            + [pltpu.VMEM((_CHUNK, 16), jnp.float32)] * 2
            + [pltpu.VMEM((_CHUNK,), jnp.float32)] * 3
            + [pltpu.SemaphoreType.DMA]),
        compiler_params=_CPARAMS,
    )
    r, g, b = gather(u, v, tbl)
    return jnp.stack([r, g, b], axis=1)


# speedup vs baseline: 36.1447x; 2.0337x over previous
import jax
import jax.numpy as jnp
from jax import lax
from jax.experimental import pallas as pl
from jax.experimental.pallas import tpu as pltpu
from jax.experimental.pallas import tpu_sc as plsc

_WIDTH = 2048
_HEIGHT = 2048
_N_UVS = 1048576
_NTEX = _HEIGHT * _WIDTH
_NELEM = _NTEX * 4
_HALF = _NELEM
_NWIN = _NELEM // 16

_NC = 2
_NS = 16
_NW = _NC * _NS
_L = 16

_SZE = _NELEM // _NW
_BB = 16384
_TB = _BB // 4
_TSTAGE = _TB + 16
_BSTEPS = _SZE // _BB
_IVLEN = _BB + 64

_CHUNK = 128
_PER_W = _N_UVS // _NW
_ITERS = _PER_W // _CHUNK


def _build_body(p0_hbm, p1_hbm, p2_hbm, tbl_hbm,
                st0a, st1a, st2a, st0b, st1b, st2b, iva, ivb,
                in_a, in_b, out_a, out_b):
    wid = lax.axis_index("s") * _NC + lax.axis_index("c")
    ebase = wid * _SZE
    lane = lax.iota(jnp.int32, _L)
    lane4 = lane * 4
    planes = (p0_hbm, p1_hbm, p2_hbm)
    stages = ((st0a, st1a, st2a), (st0b, st1b, st2b))
    ivs = (iva, ivb)
    in_sems = (in_a, in_b)
    out_sems = (out_a, out_b)

    def fire_stage(j, k):
        t0 = pl.multiple_of((ebase + j * _BB) >> 2, 8)
        for c in range(3):
            pltpu.async_copy(planes[c].at[pl.ds(t0, _TSTAGE)],
                             stages[k][c], in_sems[k])

    def wait_stage(k):
        for c in range(3):
            pltpu.make_async_copy(planes[c].at[pl.ds(0, _TSTAGE)],
                                  stages[k][c], in_sems[k]).wait()

    def fire_out(j, k):
        eb = pl.multiple_of(ebase + j * _BB, 8)
        pltpu.async_copy(ivs[k].at[pl.ds(0, _BB)],
                         tbl_hbm.at[pl.ds(eb, _BB)], out_sems[k])
        pltpu.async_copy(ivs[k].at[pl.ds(8, _BB)],
                         tbl_hbm.at[pl.ds(_HALF + eb, _BB)], out_sems[k])

    def wait_out(k):
        for _ in range(2):
            pltpu.make_async_copy(ivs[k].at[pl.ds(0, _BB)],
                                  tbl_hbm.at[pl.ds(0, _BB)],
                                  out_sems[k]).wait()

    def interleave(k):
        for g in range(_TSTAGE // _L):
            gb = 64 * g
            for c in range(3):
                val = stages[k][c][pl.ds(_L * g, _L)]
                plsc.store_scatter(ivs[k], [lane4 + (gb + c)], val)

    fire_stage(0, 0)

    def superstep(ss, carry):
        for k in range(2):
            j = ss * 2 + k
            if k == 0:
                fire_stage(j + 1, 1)
            else:
                @pl.when(ss < _BSTEPS // 2 - 1)
                def _():
                    fire_stage(j + 1, 0)
            wait_stage(k)

            @pl.when(ss >= 1)
            def _():
                wait_out(k)

            interleave(k)
            fire_out(j, k)
        return carry

    lax.fori_loop(0, _BSTEPS // 2, superstep, 0)
    wait_out(0)
    wait_out(1)


def _gather_body(u_hbm, v_hbm, tbl_hbm, r_hbm, g_hbm, b_hbm,
                 u_a, v_a, u_b, v_b,
                 rA_a, rB_a, rA_b, rB_b,
                 cb_a, cbd_a, cb_b, cbd_b, a_a, b_a, a_b, b_b,
                 cA_a, cB_a, cA_b, cB_b,
                 or_a, og_a, ob_a, or_b, og_b, ob_b,
                 in_a, in_b, g_a, g_b, out_a, out_b):
    wid = lax.axis_index("s") * _NC + lax.axis_index("c")
    base = wid * _PER_W
    lane = lax.iota(jnp.int32, _L)
    uv_bufs = ((u_a, v_a), (u_b, v_b))
    r_bufs = ((rA_a, rB_a), (rA_b, rB_b))
    col_bufs = ((cb_a, cbd_a), (cb_b, cbd_b))
    w_bufs = ((a_a, b_a), (a_b, b_b))
    c_bufs = ((cA_a, cB_a), (cA_b, cB_b))
    o_bufs = ((or_a, og_a, ob_a), (or_b, og_b, ob_b))
    in_sems = (in_a, in_b)
    g_sems = (g_a, g_b)
    out_sems = (out_a, out_b)
    out_hbms = (r_hbm, g_hbm, b_hbm)

    def fire_in(off, k):
        pltpu.async_copy(u_hbm.at[pl.ds(off, _CHUNK)], uv_bufs[k][0],
                         in_sems[k])
        pltpu.async_copy(v_hbm.at[pl.ds(off, _CHUNK)], uv_bufs[k][1],
                         in_sems[k])

    def wait_in(k):
        for c in range(2):
            pltpu.make_async_copy(u_hbm.at[pl.ds(0, _CHUNK)],
                                  uv_bufs[k][c], in_sems[k]).wait()

    def index_phase(k):
        for i in range(_CHUNK // _L):
            sl = pl.ds(_L * i, _L)
            us = uv_bufs[k][0][sl]
            vs = uv_bufs[k][1][sl]
            u = ((us + 1.0) * 0.5) * (_WIDTH - 1)
            v = ((vs + 1.0) * 0.5) * (_HEIGHT - 1)
            u0 = u.astype(jnp.int32)
            v0 = v.astype(jnp.int32)
            af = u - u0.astype(jnp.float32)
            bf = v - v0.astype(jnp.float32)
            u1 = u0 + jnp.where(af > 0.0, 1, 0)
            dv = jnp.where(bf > 0.0, 1, 0)
            s00 = u0 * _WIDTH + v0
            s10 = u1 * _WIDTH + v0
            w0 = s00 >> 1
            w1 = s10 >> 1
            cb = (s00 & 1) * 4
            r_bufs[k][0][sl] = (w0 >> 1) + ((w0 & 1) << 20)
            r_bufs[k][1][sl] = (w1 >> 1) + ((w1 & 1) << 20)
            col_bufs[k][0][sl] = cb
            col_bufs[k][1][sl] = cb + dv * 4
            w_bufs[k][0][sl] = af
            w_bufs[k][1][sl] = bf

    def fire_gathers(k):
        pltpu.async_copy(tbl_hbm.at[r_bufs[k][0]], c_bufs[k][0], g_sems[k])
        pltpu.async_copy(tbl_hbm.at[r_bufs[k][1]], c_bufs[k][1], g_sems[k])

    def wait_gathers(k):
        for c in range(2):
            pltpu.make_async_copy(tbl_hbm.at[r_bufs[k][0]],
                                  c_bufs[k][c], g_sems[k]).wait()

    def combine(k):
        cA_v, cB_v = c_bufs[k]
        for i in range(_CHUNK // _L):
            sl = pl.ds(_L * i, _L)
            af = w_bufs[k][0][sl]
            bf = w_bufs[k][1][sl]
            cb = col_bufs[k][0][sl]
            cbd = col_bufs[k][1][sl]
            naf = 1.0 - af
            nbf = 1.0 - bf
            row16 = lane + (_L * i)
            for ch in range(3):
                c00 = plsc.load_gather(cA_v, [row16, cb + ch])
                c01 = plsc.load_gather(cA_v, [row16, cbd + ch])
                c10 = plsc.load_gather(cB_v, [row16, cb + ch])
                c11 = plsc.load_gather(cB_v, [row16, cbd + ch])
                x = (c00 * af + c10 * naf) * bf + (c01 * af + c11 * naf) * nbf
                xc = jnp.minimum(jnp.maximum(x, -9.0), 9.0)
                e = jnp.exp(xc + xc)
                t = (e - 1.0) / (e + 1.0)
                o_bufs[k][ch][sl] = t

    def fire_outs(off, k):
        for ch in range(3):
            pltpu.async_copy(o_bufs[k][ch],
                             out_hbms[ch].at[pl.ds(off, _CHUNK)], out_sems[k])

    def wait_outs(k):
        for ch in range(3):
            pltpu.make_async_copy(o_bufs[k][ch],
                                  out_hbms[ch].at[pl.ds(0, _CHUNK)],
                                  out_sems[k]).wait()

    fire_in(pl.multiple_of(base, _CHUNK), 0)

    def superstep(ss, carry):
        for k in range(2):
            j = ss * 2 + k
            off = pl.multiple_of(base + j * _CHUNK, _CHUNK)
            wait_in(k)
            if k == 0:
                fire_in(off + _CHUNK, 1)
            else:
                @pl.when(ss < _ITERS // 2 - 1)
                def _():
                    fire_in(off + _CHUNK, 0)
            index_phase(k)
            fire_gathers(k)

            if k == 0:
                @pl.when(ss >= 1)
                def _():
                    wait_gathers(1)

                    @pl.when(ss >= 2)
                    def _():
                        wait_outs(1)

                    combine(1)
                    fire_outs(off - _CHUNK, 1)
            else:
                wait_gathers(0)

                @pl.when(ss >= 1)
                def _():
                    wait_outs(0)

                combine(0)
                fire_outs(off - _CHUNK, 0)
        return carry

    lax.fori_loop(0, _ITERS // 2, superstep, 0)

    last = pl.multiple_of(base + (_ITERS - 1) * _CHUNK, _CHUNK)
    wait_gathers(1)
    wait_outs(1)
    combine(1)
    fire_outs(last, 1)
    wait_outs(0)
    wait_outs(1)


_MESH = dict(core_axis_name="c", subcore_axis_name="s",
             num_cores=_NC, num_subcores=_NS)
_CPARAMS = pltpu.CompilerParams(
    needs_layout_passes=False, use_tc_tiling_on_sc=False)


def kernel(uvs, texture):
    u = uvs[:, 0]
    v = uvs[:, 1]
    planes = [jnp.pad(texture[:, :, c].reshape(-1), (0, 16))
              for c in range(3)]

    build = pl.kernel(
        _build_body,
        out_type=jax.ShapeDtypeStruct((2 * _HALF,), jnp.float32),
        mesh=plsc.VectorSubcoreMesh(**_MESH),
        scratch_types=(
            [pltpu.VMEM((_TSTAGE,), jnp.float32)] * 6
            + [pltpu.VMEM((_IVLEN,), jnp.float32)] * 2
            + [pltpu.SemaphoreType.DMA] * 4),
        compiler_params=_CPARAMS,
    )
    tbl = build(*planes).reshape(2 * _NWIN, 16)

    gather = pl.kernel(
        _gather_body,
        out_type=(jax.ShapeDtypeStruct((_N_UVS,), jnp.float32),) * 3,
        mesh=plsc.VectorSubcoreMesh(**_MESH),
        scratch_types=(
            [pltpu.VMEM((_CHUNK,), jnp.float32)] * 4
            + [pltpu.VMEM((_CHUNK,), jnp.int32)] * 4
            + [pltpu.VMEM((_CHUNK,), jnp.int32)] * 4
            + [pltpu.VMEM((_CHUNK,), jnp.float32)] * 4
            + [pltpu.VMEM((_CHUNK, 16), jnp.float32)] * 4
            + [pltpu.VMEM((_CHUNK,), jnp.float32)] * 6
            + [pltpu.SemaphoreType.DMA] * 6),
        compiler_params=_CPARAMS,
    )
    r, g, b = gather(u, v, tbl)
    return jnp.stack([r, g, b], axis=1)


# speedup vs baseline: 36.2489x; 1.0029x over previous
import jax
import jax.numpy as jnp
from jax import lax
from jax.experimental import pallas as pl
from jax.experimental.pallas import tpu as pltpu
from jax.experimental.pallas import tpu_sc as plsc

_WIDTH = 2048
_HEIGHT = 2048
_N_UVS = 1048576
_NTEX = _HEIGHT * _WIDTH
_NELEM = _NTEX * 4
_HALF = _NELEM
_NWIN = _NELEM // 16

_NC = 2
_NS = 16
_NW = _NC * _NS
_L = 16

_SZE = _NELEM // _NW
_BB = 16384
_TB = _BB // 4
_TSTAGE = _TB + 16
_BSTEPS = _SZE // _BB
_IVLEN = _BB + 64

_CHUNK = 128
_SCH = 1024
_QS = _SCH // _CHUNK
_PER_W = _N_UVS // _NW
_ITERS = _PER_W // _SCH


def _build_body(p0_hbm, p1_hbm, p2_hbm, tbl_hbm,
                st0a, st1a, st2a, st0b, st1b, st2b, iva, ivb,
                in_a, in_b, out_a, out_b):
    wid = lax.axis_index("s") * _NC + lax.axis_index("c")
    ebase = wid * _SZE
    lane = lax.iota(jnp.int32, _L)
    lane4 = lane * 4
    planes = (p0_hbm, p1_hbm, p2_hbm)
    stages = ((st0a, st1a, st2a), (st0b, st1b, st2b))
    ivs = (iva, ivb)
    in_sems = (in_a, in_b)
    out_sems = (out_a, out_b)

    def fire_stage(j, k):
        t0 = pl.multiple_of((ebase + j * _BB) >> 2, 8)
        for c in range(3):
            pltpu.async_copy(planes[c].at[pl.ds(t0, _TSTAGE)],
                             stages[k][c], in_sems[k])

    def wait_stage(k):
        for c in range(3):
            pltpu.make_async_copy(planes[c].at[pl.ds(0, _TSTAGE)],
                                  stages[k][c], in_sems[k]).wait()

    def fire_out(j, k):
        eb = pl.multiple_of(ebase + j * _BB, 8)
        pltpu.async_copy(ivs[k].at[pl.ds(0, _BB)],
                         tbl_hbm.at[pl.ds(eb, _BB)], out_sems[k])
        pltpu.async_copy(ivs[k].at[pl.ds(8, _BB)],
                         tbl_hbm.at[pl.ds(_HALF + eb, _BB)], out_sems[k])

    def wait_out(k):
        for _ in range(2):
            pltpu.make_async_copy(ivs[k].at[pl.ds(0, _BB)],
                                  tbl_hbm.at[pl.ds(0, _BB)],
                                  out_sems[k]).wait()

    def interleave(k):
        for g in range(_TSTAGE // _L):
            gb = 64 * g
            for c in range(3):
                val = stages[k][c][pl.ds(_L * g, _L)]
                plsc.store_scatter(ivs[k], [lane4 + (gb + c)], val)

    fire_stage(0, 0)

    def superstep(ss, carry):
        for k in range(2):
            j = ss * 2 + k
            if k == 0:
                fire_stage(j + 1, 1)
            else:
                @pl.when(ss < _BSTEPS // 2 - 1)
                def _():
                    fire_stage(j + 1, 0)
            wait_stage(k)

            @pl.when(ss >= 1)
            def _():
                wait_out(k)

            interleave(k)
            fire_out(j, k)
        return carry

    lax.fori_loop(0, _BSTEPS // 2, superstep, 0)
    wait_out(0)
    wait_out(1)


def _gather_body(u_hbm, v_hbm, tbl_hbm, r_hbm, g_hbm, b_hbm,
                 u_a, v_a, u_b, v_b,
                 rA_a, rB_a, rA_b, rB_b,
                 cb_a, cbd_a, cb_b, cbd_b, a_a, b_a, a_b, b_b,
                 cA_a, cB_a, cA_b, cB_b,
                 or_a, og_a, ob_a, or_b, og_b, ob_b,
                 in_a, in_b, g_a, g_b, out_a, out_b):
    wid = lax.axis_index("s") * _NC + lax.axis_index("c")
    base = wid * _PER_W
    lane = lax.iota(jnp.int32, _L)
    uv_bufs = ((u_a, v_a), (u_b, v_b))
    r_bufs = ((rA_a, rB_a), (rA_b, rB_b))
    col_bufs = ((cb_a, cbd_a), (cb_b, cbd_b))
    w_bufs = ((a_a, b_a), (a_b, b_b))
    c_bufs = ((cA_a, cB_a), (cA_b, cB_b))
    o_bufs = ((or_a, og_a, ob_a), (or_b, og_b, ob_b))
    in_sems = (in_a, in_b)
    g_sems = (g_a, g_b)
    out_sems = (out_a, out_b)
    out_hbms = (r_hbm, g_hbm, b_hbm)

    def fire_in(off, k):
        pltpu.async_copy(u_hbm.at[pl.ds(off, _SCH)], uv_bufs[k][0],
                         in_sems[k])
        pltpu.async_copy(v_hbm.at[pl.ds(off, _SCH)], uv_bufs[k][1],
                         in_sems[k])

    def wait_in(k):
        for c in range(2):
            pltpu.make_async_copy(u_hbm.at[pl.ds(0, _SCH)],
                                  uv_bufs[k][c], in_sems[k]).wait()

    def index_phase(k):
        def body(i, carry):
            sl = pl.ds(pl.multiple_of(_L * i, _L), _L)
            us = uv_bufs[k][0][sl]
            vs = uv_bufs[k][1][sl]
            u = ((us + 1.0) * 0.5) * (_WIDTH - 1)
            v = ((vs + 1.0) * 0.5) * (_HEIGHT - 1)
            u0 = u.astype(jnp.int32)
            v0 = v.astype(jnp.int32)
            af = u - u0.astype(jnp.float32)
            bf = v - v0.astype(jnp.float32)
            u1 = u0 + jnp.where(af > 0.0, 1, 0)
            dv = jnp.where(bf > 0.0, 1, 0)
            s00 = u0 * _WIDTH + v0
            s10 = u1 * _WIDTH + v0
            w0 = s00 >> 1
            w1 = s10 >> 1
            cb = (s00 & 1) * 4
            r_bufs[k][0][sl] = (w0 >> 1) + ((w0 & 1) << 20)
            r_bufs[k][1][sl] = (w1 >> 1) + ((w1 & 1) << 20)
            col_bufs[k][0][sl] = cb
            col_bufs[k][1][sl] = cb + dv * 4
            w_bufs[k][0][sl] = af
            w_bufs[k][1][sl] = bf
            return carry

        lax.fori_loop(0, _SCH // _L, body, 0)

    def fire_gathers(k):
        for c in range(2):
            for q in range(_QS):
                pltpu.async_copy(
                    tbl_hbm.at[r_bufs[k][c].at[pl.ds(_CHUNK * q, _CHUNK)]],
                    c_bufs[k][c].at[pl.ds(_CHUNK * q, _CHUNK)], g_sems[k])

    def wait_gathers(k):
        for c in range(2):
            for q in range(_QS):
                pltpu.make_async_copy(
                    tbl_hbm.at[r_bufs[k][0].at[pl.ds(0, _CHUNK)]],
                    c_bufs[k][c].at[pl.ds(_CHUNK * q, _CHUNK)],
                    g_sems[k]).wait()

    def combine(k):
        cA_v, cB_v = c_bufs[k]

        def body(i, carry):
            sl = pl.ds(pl.multiple_of(_L * i, _L), _L)
            af = w_bufs[k][0][sl]
            bf = w_bufs[k][1][sl]
            cb = col_bufs[k][0][sl]
            cbd = col_bufs[k][1][sl]
            naf = 1.0 - af
            nbf = 1.0 - bf
            row16 = lane + (_L * i)
            for ch in range(3):
                c00 = plsc.load_gather(cA_v, [row16, cb + ch])
                c01 = plsc.load_gather(cA_v, [row16, cbd + ch])
                c10 = plsc.load_gather(cB_v, [row16, cb + ch])
                c11 = plsc.load_gather(cB_v, [row16, cbd + ch])
                x = (c00 * af + c10 * naf) * bf + (c01 * af + c11 * naf) * nbf
                xc = jnp.minimum(jnp.maximum(x, -9.0), 9.0)
                e = jnp.exp(xc + xc)
                t = (e - 1.0) / (e + 1.0)
                o_bufs[k][ch][sl] = t
            return carry

        lax.fori_loop(0, _SCH // _L, body, 0)

    def fire_outs(off, k):
        for ch in range(3):
            pltpu.async_copy(o_bufs[k][ch],
                             out_hbms[ch].at[pl.ds(off, _SCH)], out_sems[k])

    def wait_outs(k):
        for ch in range(3):
            pltpu.make_async_copy(o_bufs[k][ch],
                                  out_hbms[ch].at[pl.ds(0, _SCH)],
                                  out_sems[k]).wait()

    fire_in(pl.multiple_of(base, _SCH), 0)

    def superstep(ss, carry):
        for k in range(2):
            j = ss * 2 + k
            off = pl.multiple_of(base + j * _SCH, _SCH)
            wait_in(k)
            if k == 0:
                fire_in(off + _SCH, 1)
            else:
                @pl.when(ss < _ITERS // 2 - 1)
                def _():
                    fire_in(off + _SCH, 0)
            index_phase(k)
            fire_gathers(k)

            if k == 0:
                @pl.when(ss >= 1)
                def _():
                    wait_gathers(1)

                    @pl.when(ss >= 2)
                    def _():
                        wait_outs(1)

                    combine(1)
                    fire_outs(off - _SCH, 1)
            else:
                wait_gathers(0)

                @pl.when(ss >= 1)
                def _():
                    wait_outs(0)

                combine(0)
                fire_outs(off - _SCH, 0)
        return carry

    lax.fori_loop(0, _ITERS // 2, superstep, 0)

    last = pl.multiple_of(base + (_ITERS - 1) * _SCH, _SCH)
    wait_gathers(1)
    wait_outs(1)
    combine(1)
    fire_outs(last, 1)
    wait_outs(0)
    wait_outs(1)


_MESH = dict(core_axis_name="c", subcore_axis_name="s",
             num_cores=_NC, num_subcores=_NS)
_CPARAMS = pltpu.CompilerParams(
    needs_layout_passes=False, use_tc_tiling_on_sc=False)


def kernel(uvs, texture):
    u = uvs[:, 0]
    v = uvs[:, 1]
    planes = [jnp.pad(texture[:, :, c].reshape(-1), (0, 16))
              for c in range(3)]

    build = pl.kernel(
        _build_body,
        out_type=jax.ShapeDtypeStruct((2 * _HALF,), jnp.float32),
        mesh=plsc.VectorSubcoreMesh(**_MESH),
        scratch_types=(
            [pltpu.VMEM((_TSTAGE,), jnp.float32)] * 6
            + [pltpu.VMEM((_IVLEN,), jnp.float32)] * 2
            + [pltpu.SemaphoreType.DMA] * 4),
        compiler_params=_CPARAMS,
    )
    tbl = build(*planes).reshape(2 * _NWIN, 16)

    gather = pl.kernel(
        _gather_body,
        out_type=(jax.ShapeDtypeStruct((_N_UVS,), jnp.float32),) * 3,
        mesh=plsc.VectorSubcoreMesh(**_MESH),
        scratch_types=(
            [pltpu.VMEM((_SCH,), jnp.float32)] * 4
            + [pltpu.VMEM((_SCH,), jnp.int32)] * 4
            + [pltpu.VMEM((_SCH,), jnp.int32)] * 4
            + [pltpu.VMEM((_SCH,), jnp.float32)] * 4
            + [pltpu.VMEM((_SCH, 16), jnp.float32)] * 4
            + [pltpu.VMEM((_SCH,), jnp.float32)] * 6
            + [pltpu.SemaphoreType.DMA] * 6),
        compiler_params=_CPARAMS,
    )
    r, g, b = gather(u, v, tbl)
    return jnp.stack([r, g, b], axis=1)


# speedup vs baseline: 36.8975x; 1.0179x over previous
import jax
import jax.numpy as jnp
from jax import lax
from jax.experimental import pallas as pl
from jax.experimental.pallas import tpu as pltpu
from jax.experimental.pallas import tpu_sc as plsc

_WIDTH = 2048
_HEIGHT = 2048
_N_UVS = 1048576
_NTEX = _HEIGHT * _WIDTH
_NELEM = _NTEX * 4
_HALF = _NELEM
_NWIN = _NELEM // 16

_NC = 2
_NS = 16
_NW = _NC * _NS
_L = 16

_SZE = _NELEM // _NW
_BB = 16384
_TB = _BB // 4
_TSTAGE = _TB + 16
_BSTEPS = _SZE // _BB
_IVLEN = _BB + 64

_CHUNK = 128
_SCH = 1024
_QS = _SCH // _CHUNK
_PER_W = _N_UVS // _NW
_ITERS = _PER_W // _SCH


def _build_body(p0_hbm, p1_hbm, p2_hbm, tbl_hbm,
                st0a, st1a, st2a, st0b, st1b, st2b, iva, ivb,
                in_a, in_b, out_a, out_b):
    wid = lax.axis_index("s") * _NC + lax.axis_index("c")
    ebase = wid * _SZE
    lane = lax.iota(jnp.int32, _L)
    lane4 = lane * 4
    planes = (p0_hbm, p1_hbm, p2_hbm)
    stages = ((st0a, st1a, st2a), (st0b, st1b, st2b))
    ivs = (iva, ivb)
    in_sems = (in_a, in_b)
    out_sems = (out_a, out_b)

    def fire_stage(j, k):
        t0 = pl.multiple_of((ebase + j * _BB) >> 2, 8)
        for c in range(3):
            pltpu.async_copy(planes[c].at[pl.ds(t0, _TSTAGE)],
                             stages[k][c], in_sems[k])

    def wait_stage(k):
        for c in range(3):
            pltpu.make_async_copy(planes[c].at[pl.ds(0, _TSTAGE)],
                                  stages[k][c], in_sems[k]).wait()

    def fire_out(j, k):
        eb = pl.multiple_of(ebase + j * _BB, 8)
        pltpu.async_copy(ivs[k].at[pl.ds(0, _BB)],
                         tbl_hbm.at[pl.ds(eb, _BB)], out_sems[k])
        pltpu.async_copy(ivs[k].at[pl.ds(8, _BB)],
                         tbl_hbm.at[pl.ds(_HALF + eb, _BB)], out_sems[k])

    def wait_out(k):
        for _ in range(2):
            pltpu.make_async_copy(ivs[k].at[pl.ds(0, _BB)],
                                  tbl_hbm.at[pl.ds(0, _BB)],
                                  out_sems[k]).wait()

    def interleave(k):
        for g in range(_TSTAGE // _L):
            gb = 64 * g
            for c in range(3):
                val = stages[k][c][pl.ds(_L * g, _L)]
                plsc.store_scatter(ivs[k], [lane4 + (gb + c)], val)

    fire_stage(0, 0)

    def superstep(ss, carry):
        for k in range(2):
            j = ss * 2 + k
            if k == 0:
                fire_stage(j + 1, 1)
            else:
                @pl.when(ss < _BSTEPS // 2 - 1)
                def _():
                    fire_stage(j + 1, 0)
            wait_stage(k)

            @pl.when(ss >= 1)
            def _():
                wait_out(k)

            interleave(k)
            fire_out(j, k)
        return carry

    lax.fori_loop(0, _BSTEPS // 2, superstep, 0)
    wait_out(0)
    wait_out(1)


def _gather_body(u_hbm, v_hbm, tbl_hbm, r_hbm, g_hbm, b_hbm,
                 u_a, v_a, u_b, v_b,
                 rA_a, rB_a, rA_b, rB_b,
                 cb_a, cbd_a, cb_b, cbd_b, a_a, b_a, a_b, b_b,
                 cA_a, cB_a, cA_b, cB_b,
                 or_a, og_a, ob_a, or_b, og_b, ob_b,
                 in_a, in_b, g_a, g_b, out_a, out_b):
    wid = lax.axis_index("s") * _NC + lax.axis_index("c")
    base = wid * _PER_W
    lane = lax.iota(jnp.int32, _L)
    uv_bufs = ((u_a, v_a), (u_b, v_b))
    r_bufs = ((rA_a, rB_a), (rA_b, rB_b))
    col_bufs = ((cb_a, cbd_a), (cb_b, cbd_b))
    w_bufs = ((a_a, b_a), (a_b, b_b))
    c_bufs = ((cA_a, cB_a), (cA_b, cB_b))
    o_bufs = ((or_a, og_a, ob_a), (or_b, og_b, ob_b))
    in_sems = (in_a, in_b)
    g_sems = (g_a, g_b)
    out_sems = (out_a, out_b)
    out_hbms = (r_hbm, g_hbm, b_hbm)

    def fire_in(off, k):
        pltpu.async_copy(u_hbm.at[pl.ds(off, _SCH)], uv_bufs[k][0],
                         in_sems[k])
        pltpu.async_copy(v_hbm.at[pl.ds(off, _SCH)], uv_bufs[k][1],
                         in_sems[k])

    def wait_in(k):
        for c in range(2):
            pltpu.make_async_copy(u_hbm.at[pl.ds(0, _SCH)],
                                  uv_bufs[k][c], in_sems[k]).wait()

    def index_phase(k):
        def body(i2, carry):
            for t in range(2):
                i = i2 * 2 + t
                sl = pl.ds(pl.multiple_of(_L * i, _L), _L)
                us = uv_bufs[k][0][sl]
                vs = uv_bufs[k][1][sl]
                u = ((us + 1.0) * 0.5) * (_WIDTH - 1)
                v = ((vs + 1.0) * 0.5) * (_HEIGHT - 1)
                u0 = u.astype(jnp.int32)
                v0 = v.astype(jnp.int32)
                af = u - u0.astype(jnp.float32)
                bf = v - v0.astype(jnp.float32)
                u1 = u0 + jnp.where(af > 0.0, 1, 0)
                dv = jnp.where(bf > 0.0, 1, 0)
                s00 = u0 * _WIDTH + v0
                s10 = u1 * _WIDTH + v0
                w0 = s00 >> 1
                w1 = s10 >> 1
                cb = (s00 & 1) * 4
                r_bufs[k][0][sl] = (w0 >> 1) + ((w0 & 1) << 20)
                r_bufs[k][1][sl] = (w1 >> 1) + ((w1 & 1) << 20)
                col_bufs[k][0][sl] = cb
                col_bufs[k][1][sl] = cb + dv * 4
                w_bufs[k][0][sl] = af
                w_bufs[k][1][sl] = bf
            return carry

        lax.fori_loop(0, _SCH // _L // 2, body, 0)

    def fire_gathers(k):
        for c in range(2):
            for q in range(_QS):
                pltpu.async_copy(
                    tbl_hbm.at[r_bufs[k][c].at[pl.ds(_CHUNK * q, _CHUNK)]],
                    c_bufs[k][c].at[pl.ds(_CHUNK * q, _CHUNK)], g_sems[k])

    def wait_gathers(k):
        for c in range(2):
            for q in range(_QS):
                pltpu.make_async_copy(
                    tbl_hbm.at[r_bufs[k][0].at[pl.ds(0, _CHUNK)]],
                    c_bufs[k][c].at[pl.ds(_CHUNK * q, _CHUNK)],
                    g_sems[k]).wait()

    def combine(k):
        cA_v, cB_v = c_bufs[k]

        def body(i2, carry):
            for t in range(2):
                i = i2 * 2 + t
                sl = pl.ds(pl.multiple_of(_L * i, _L), _L)
                af = w_bufs[k][0][sl]
                bf = w_bufs[k][1][sl]
                cb = col_bufs[k][0][sl]
                cbd = col_bufs[k][1][sl]
                naf = 1.0 - af
                nbf = 1.0 - bf
                row16 = lane + (_L * i)
                for ch in range(3):
                    c00 = plsc.load_gather(cA_v, [row16, cb + ch])
                    c01 = plsc.load_gather(cA_v, [row16, cbd + ch])
                    c10 = plsc.load_gather(cB_v, [row16, cb + ch])
                    c11 = plsc.load_gather(cB_v, [row16, cbd + ch])
                    x = ((c00 * af + c10 * naf) * bf
                         + (c01 * af + c11 * naf) * nbf)
                    xc = jnp.minimum(jnp.maximum(x, -9.0), 9.0)
                    e = jnp.exp(xc + xc)
                    tt = (e - 1.0) / (e + 1.0)
                    o_bufs[k][ch][sl] = tt
            return carry

        lax.fori_loop(0, _SCH // _L // 2, body, 0)

    def fire_outs(off, k):
        for ch in range(3):
            pltpu.async_copy(o_bufs[k][ch],
                             out_hbms[ch].at[pl.ds(off, _SCH)], out_sems[k])

    def wait_outs(k):
        for ch in range(3):
            pltpu.make_async_copy(o_bufs[k][ch],
                                  out_hbms[ch].at[pl.ds(0, _SCH)],
                                  out_sems[k]).wait()

    fire_in(pl.multiple_of(base, _SCH), 0)

    def superstep(ss, carry):
        for k in range(2):
            j = ss * 2 + k
            off = pl.multiple_of(base + j * _SCH, _SCH)
            wait_in(k)
            if k == 0:
                fire_in(off + _SCH, 1)
            else:
                @pl.when(ss < _ITERS // 2 - 1)
                def _():
                    fire_in(off + _SCH, 0)
            index_phase(k)
            fire_gathers(k)

            if k == 0:
                @pl.when(ss >= 1)
                def _():
                    wait_gathers(1)

                    @pl.when(ss >= 2)
                    def _():
                        wait_outs(1)

                    combine(1)
                    fire_outs(off - _SCH, 1)
            else:
                wait_gathers(0)

                @pl.when(ss >= 1)
                def _():
                    wait_outs(0)

                combine(0)
                fire_outs(off - _SCH, 0)
        return carry

    lax.fori_loop(0, _ITERS // 2, superstep, 0)

    last = pl.multiple_of(base + (_ITERS - 1) * _SCH, _SCH)
    wait_gathers(1)
    wait_outs(1)
    combine(1)
    fire_outs(last, 1)
    wait_outs(0)
    wait_outs(1)


_MESH = dict(core_axis_name="c", subcore_axis_name="s",
             num_cores=_NC, num_subcores=_NS)
_CPARAMS = pltpu.CompilerParams(
    needs_layout_passes=False, use_tc_tiling_on_sc=False)


def kernel(uvs, texture):
    u = uvs[:, 0]
    v = uvs[:, 1]
    planes = [jnp.pad(texture[:, :, c].reshape(-1), (0, 16))
              for c in range(3)]

    build = pl.kernel(
        _build_body,
        out_type=jax.ShapeDtypeStruct((2 * _HALF,), jnp.float32),
        mesh=plsc.VectorSubcoreMesh(**_MESH),
        scratch_types=(
            [pltpu.VMEM((_TSTAGE,), jnp.float32)] * 6
            + [pltpu.VMEM((_IVLEN,), jnp.float32)] * 2
            + [pltpu.SemaphoreType.DMA] * 4),
        compiler_params=_CPARAMS,
    )
    tbl = build(*planes).reshape(2 * _NWIN, 16)

    gather = pl.kernel(
        _gather_body,
        out_type=(jax.ShapeDtypeStruct((_N_UVS,), jnp.float32),) * 3,
        mesh=plsc.VectorSubcoreMesh(**_MESH),
        scratch_types=(
            [pltpu.VMEM((_SCH,), jnp.float32)] * 4
            + [pltpu.VMEM((_SCH,), jnp.int32)] * 4
            + [pltpu.VMEM((_SCH,), jnp.int32)] * 4
            + [pltpu.VMEM((_SCH,), jnp.float32)] * 4
            + [pltpu.VMEM((_SCH, 16), jnp.float32)] * 4
            + [pltpu.VMEM((_SCH,), jnp.float32)] * 6
            + [pltpu.SemaphoreType.DMA] * 6),
        compiler_params=_CPARAMS,
    )
    r, g, b = gather(u, v, tbl)
    return jnp.stack([r, g, b], axis=1)
